# Initial kernel scaffold; baseline (speedup 1.0000x reference)
import functools

import jax
import jax.numpy as jnp
from jax import lax
from jax.experimental import pallas as pl
from jax.experimental.pallas import tpu as pltpu
from jax.experimental.pallas import tpu_sc as plsc

ALPHA = 0.1
K_ITERS = 10

NC = 2
NS = 16
NW = NC * NS
WIN = 128


def _sc_scatter(table, srcp, dstp, wp, zeros_pad, n_pad, c_dim, nwin):
    spt = n_pad // NS

    def body(table_ref, src_ref, dst_ref, w_ref, z_ref, out_ref,
             acc, src_v, dst_v, w_v, rows_v, sem):
        c = lax.axis_index("c")
        s = lax.axis_index("s")
        wid = c * NS + s
        pltpu.sync_copy(src_ref.at[wid], src_v)
        pltpu.sync_copy(dst_ref.at[wid], dst_v)
        pltpu.sync_copy(w_ref.at[wid], w_v)
        node0 = s * spt
        pltpu.sync_copy(z_ref.at[pl.ds(node0, spt)], acc.at[pl.ds(node0, spt)])
        plsc.subcore_barrier()

        def win(j, carry):
            pltpu.async_copy(table_ref.at[src_v.at[j]], rows_v, sem).wait()

            def edge(e, carry2):
                sc = w_v[j, e]
                for q in range(c_dim // 16):
                    rows_v[e, pl.ds(q * 16, 16)] = rows_v[e, pl.ds(q * 16, 16)] * sc
                return carry2

            lax.fori_loop(0, WIN, edge, 0, unroll=2)
            pltpu.async_copy(rows_v, acc.at[dst_v.at[j]], sem, add=True).wait()
            return carry

        lax.fori_loop(0, nwin, win, 0)
        plsc.subcore_barrier()
        pltpu.sync_copy(acc.at[pl.ds(node0, spt)], out_ref.at[c].at[pl.ds(node0, spt)])

    mesh = plsc.VectorSubcoreMesh(core_axis_name="c", subcore_axis_name="s")
    f = pl.kernel(
        body,
        out_type=jax.ShapeDtypeStruct((NC, n_pad, c_dim), jnp.float32),
        mesh=mesh,
        scratch_types=[
            pltpu.VMEM_SHARED((n_pad, c_dim), jnp.float32),
            pltpu.VMEM((nwin, WIN), jnp.int32),
            pltpu.VMEM((nwin, WIN), jnp.int32),
            pltpu.VMEM((nwin, WIN), jnp.float32),
            pltpu.VMEM((WIN, c_dim), jnp.float32),
            pltpu.SemaphoreType.DMA,
        ],
    )
    return f(table, srcp, dstp, wp, zeros_pad)


def _mlp(x, W1, b1, W2, b2, n_pad, blk):
    f_in = x.shape[1]
    c_dim = W2.shape[0]

    def body(x_ref, w1_ref, b1_ref, w2_ref, b2_ref, o_ref):
        h = jnp.maximum(
            jnp.dot(x_ref[...], w1_ref[...].T, preferred_element_type=jnp.float32)
            + b1_ref[...], 0.0)
        o_ref[...] = (jnp.dot(h, w2_ref[...].T, preferred_element_type=jnp.float32)
                      + b2_ref[...])

    grid = n_pad // blk
    return pl.pallas_call(
        body,
        grid=(grid,),
        in_specs=[
            pl.BlockSpec((blk, f_in), lambda i: (i, 0)),
            pl.BlockSpec(W1.shape, lambda i: (0, 0)),
            pl.BlockSpec((1, W1.shape[0]), lambda i: (0, 0)),
            pl.BlockSpec(W2.shape, lambda i: (0, 0)),
            pl.BlockSpec((1, c_dim), lambda i: (0, 0)),
        ],
        out_specs=pl.BlockSpec((blk, c_dim), lambda i: (i, 0)),
        out_shape=jax.ShapeDtypeStruct((n_pad, c_dim), jnp.float32),
    )(x, W1, b1.reshape(1, -1), W2, b2.reshape(1, -1))


def _prep(parts, h0, n_pad, blk):
    c_dim = h0.shape[1]

    def body(p_ref, h0_ref, dinv_ref, hs_ref):
        deg = p_ref[0, :, 0:1] + p_ref[1, :, 0:1] + 1.0
        dinv = lax.rsqrt(deg)
        dinv_ref[...] = jnp.broadcast_to(dinv, (blk, c_dim))
        hs_ref[...] = dinv * h0_ref[...]

    grid = n_pad // blk
    return pl.pallas_call(
        body,
        grid=(grid,),
        in_specs=[
            pl.BlockSpec((NC, blk, c_dim), lambda i: (0, i, 0)),
            pl.BlockSpec((blk, c_dim), lambda i: (i, 0)),
        ],
        out_specs=[
            pl.BlockSpec((blk, c_dim), lambda i: (i, 0)),
            pl.BlockSpec((blk, c_dim), lambda i: (i, 0)),
        ],
        out_shape=[
            jax.ShapeDtypeStruct((n_pad, c_dim), jnp.float32),
            jax.ShapeDtypeStruct((n_pad, c_dim), jnp.float32),
        ],
    )(parts, h0)


def _update(parts, hs, h0, dinv, n_pad, blk):
    c_dim = h0.shape[1]

    def body(p_ref, hs_ref, h0_ref, dinv_ref, h_ref, hsn_ref):
        t = p_ref[0] + p_ref[1] + hs_ref[...]
        h_new = (1.0 - ALPHA) * dinv_ref[...] * t + ALPHA * h0_ref[...]
        h_ref[...] = h_new
        hsn_ref[...] = dinv_ref[...] * h_new

    grid = n_pad // blk
    spec = pl.BlockSpec((blk, c_dim), lambda i: (i, 0))
    return pl.pallas_call(
        body,
        grid=(grid,),
        in_specs=[pl.BlockSpec((NC, blk, c_dim), lambda i: (0, i, 0)),
                  spec, spec, spec],
        out_specs=[spec, spec],
        out_shape=[jax.ShapeDtypeStruct((n_pad, c_dim), jnp.float32),
                   jax.ShapeDtypeStruct((n_pad, c_dim), jnp.float32)],
    )(parts, hs, h0, dinv)


def _log_softmax(h, blk):
    n, c_dim = h.shape

    def body(h_ref, o_ref):
        v = h_ref[...]
        m = jnp.max(v, axis=1, keepdims=True)
        e = jnp.exp(v - m)
        s = jnp.sum(e, axis=1, keepdims=True)
        o_ref[...] = v - m - jnp.log(s)

    grid = n // blk
    spec = pl.BlockSpec((blk, c_dim), lambda i: (i, 0))
    return pl.pallas_call(
        body,
        grid=(grid,),
        in_specs=[spec],
        out_specs=spec,
        out_shape=jax.ShapeDtypeStruct((n, c_dim), jnp.float32),
    )(h)


def kernel(x, edge_index, edge_attr, W1, b1, W2, b2):
    n, f_in = x.shape
    e_tot = edge_attr.shape[0]
    c_dim = W2.shape[0]

    n_pad = ((n + 16 * 64 - 1) // (16 * 64)) * (16 * 64)
    epw = e_tot // NW
    epw_pad = ((epw + WIN - 1) // WIN) * WIN
    nwin = epw_pad // WIN
    pad = epw_pad - epw

    src = edge_index[0].reshape(NW, epw)
    dst = edge_index[1].reshape(NW, epw)
    w = edge_attr.reshape(NW, epw)
    pad_ids = (jnp.arange(pad, dtype=jnp.int32) * 89) % n
    pad_blk = jnp.broadcast_to(pad_ids, (NW, pad))
    srcp = jnp.concatenate([src, pad_blk], axis=1).reshape(NW, nwin, WIN)
    dstp = jnp.concatenate([dst, pad_blk], axis=1).reshape(NW, nwin, WIN)
    wp = jnp.concatenate([w, jnp.zeros((NW, pad), jnp.float32)], axis=1)
    wp = wp.reshape(NW, nwin, WIN)

    zeros_pad = jnp.zeros((n_pad, c_dim), jnp.float32)
    ones_tab = jnp.ones((n_pad, c_dim), jnp.float32)

    x_pad = jnp.pad(x, ((0, n_pad - n), (0, 0)))

    blk = n_pad // 10

    deg_parts = _sc_scatter(ones_tab, srcp, dstp, wp, zeros_pad, n_pad, c_dim, nwin)
    h0 = _mlp(x_pad, W1, b1, W2, b2, n_pad, blk)
    dinv, hs = _prep(deg_parts, h0, n_pad, blk)

    h = h0
    for _ in range(K_ITERS):
        parts = _sc_scatter(hs, srcp, dstp, wp, zeros_pad, n_pad, c_dim, nwin)
        h, hs = _update(parts, hs, h0, dinv, n_pad, blk)

    return _log_softmax(h[:n], 1000)


# speedup vs baseline: 10.9772x; 10.9772x over previous
import jax
import jax.numpy as jnp
from jax import lax
from jax.experimental import pallas as pl
from jax.experimental.pallas import tpu as pltpu
from jax.experimental.pallas import tpu_sc as plsc

ALPHA = 0.1
K_ITERS = 10

NC = 2
NS = 16
NW = NC * NS
WIN = 128
CP = 128
CV = 64


def _sc_scatter(table, srcp, dstp, wp, zeros_pad, n_pad, nwin):
    spt = n_pad // NS

    def body(table_ref, src_ref, dst_ref, w_ref, z_ref, out_ref,
             acc, src_v, dst_v, w_v, rows_v, src_win, dst_win, sem):
        c = lax.axis_index("c")
        s = lax.axis_index("s")
        wid = c * NS + s
        pltpu.sync_copy(src_ref.at[wid], src_v)
        pltpu.sync_copy(dst_ref.at[wid], dst_v)
        pltpu.sync_copy(w_ref.at[wid], w_v)
        node0 = s * spt
        pltpu.sync_copy(z_ref.at[pl.ds(node0, spt)], acc.at[pl.ds(node0, spt)])
        plsc.subcore_barrier()

        def win(j, carry):
            for i in range(WIN // 16):
                src_win[pl.ds(i * 16, 16)] = src_v[j, pl.ds(i * 16, 16)]
                dst_win[pl.ds(i * 16, 16)] = dst_v[j, pl.ds(i * 16, 16)]
            pltpu.async_copy(table_ref.at[src_win], rows_v, sem).wait()

            def grp(g, carry2):
                wv = w_v[j, pl.ds(g * 16, 16)]
                for i in range(16):
                    e = g * 16 + i
                    sc = wv[i]
                    for q in range(CV // 16):
                        rows_v[e, pl.ds(q * 16, 16)] = (
                            rows_v[e, pl.ds(q * 16, 16)] * sc)
                return carry2

            lax.fori_loop(0, WIN // 16, grp, 0)
            pltpu.async_copy(rows_v, acc.at[dst_win], sem, add=True).wait()
            return carry

        lax.fori_loop(0, nwin, win, 0)
        plsc.subcore_barrier()
        pltpu.sync_copy(acc.at[pl.ds(node0, spt)], out_ref.at[c].at[pl.ds(node0, spt)])

    mesh = plsc.VectorSubcoreMesh(core_axis_name="c", subcore_axis_name="s")
    f = pl.kernel(
        body,
        out_type=jax.ShapeDtypeStruct((NC, n_pad, CP), jnp.float32),
        mesh=mesh,
        scratch_types=[
            pltpu.VMEM_SHARED((n_pad, CP), jnp.float32),
            pltpu.VMEM((nwin, WIN), jnp.int32),
            pltpu.VMEM((nwin, WIN), jnp.int32),
            pltpu.VMEM((nwin, WIN), jnp.float32),
            pltpu.VMEM((WIN, CP), jnp.float32),
            pltpu.VMEM((WIN,), jnp.int32),
            pltpu.VMEM((WIN,), jnp.int32),
            pltpu.SemaphoreType.DMA,
        ],
    )
    return f(table, srcp, dstp, wp, zeros_pad)


def _mlp(x, W1, b1, W2p, b2p, n_pad, blk):
    f_in = x.shape[1]

    def body(x_ref, w1_ref, b1_ref, w2_ref, b2_ref, o_ref):
        h = jnp.maximum(
            jnp.dot(x_ref[...], w1_ref[...].T, preferred_element_type=jnp.float32)
            + b1_ref[...], 0.0)
        o_ref[...] = (jnp.dot(h, w2_ref[...].T, preferred_element_type=jnp.float32)
                      + b2_ref[...])

    grid = n_pad // blk
    return pl.pallas_call(
        body,
        grid=(grid,),
        in_specs=[
            pl.BlockSpec((blk, f_in), lambda i: (i, 0)),
            pl.BlockSpec(W1.shape, lambda i: (0, 0)),
            pl.BlockSpec((1, W1.shape[0]), lambda i: (0, 0)),
            pl.BlockSpec(W2p.shape, lambda i: (0, 0)),
            pl.BlockSpec((1, CP), lambda i: (0, 0)),
        ],
        out_specs=pl.BlockSpec((blk, CP), lambda i: (i, 0)),
        out_shape=jax.ShapeDtypeStruct((n_pad, CP), jnp.float32),
    )(x, W1, b1.reshape(1, -1), W2p, b2p.reshape(1, -1))


def _prep(parts, h0, n_pad, blk):
    def body(p_ref, h0_ref, dinv_ref, hs_ref):
        deg = p_ref[0, :, 0:1] + p_ref[1, :, 0:1] + 1.0
        dinv = lax.rsqrt(deg)
        dinv_ref[...] = jnp.broadcast_to(dinv, (blk, CP))
        hs_ref[...] = dinv * h0_ref[...]

    grid = n_pad // blk
    spec = pl.BlockSpec((blk, CP), lambda i: (i, 0))
    return pl.pallas_call(
        body,
        grid=(grid,),
        in_specs=[pl.BlockSpec((NC, blk, CP), lambda i: (0, i, 0)), spec],
        out_specs=[spec, spec],
        out_shape=[jax.ShapeDtypeStruct((n_pad, CP), jnp.float32),
                   jax.ShapeDtypeStruct((n_pad, CP), jnp.float32)],
    )(parts, h0)


def _update(parts, hs, h0, dinv, n_pad, blk):
    def body(p_ref, hs_ref, h0_ref, dinv_ref, h_ref, hsn_ref):
        t = p_ref[0] + p_ref[1] + hs_ref[...]
        h_new = (1.0 - ALPHA) * dinv_ref[...] * t + ALPHA * h0_ref[...]
        h_ref[...] = h_new
        hsn_ref[...] = dinv_ref[...] * h_new

    grid = n_pad // blk
    spec = pl.BlockSpec((blk, CP), lambda i: (i, 0))
    return pl.pallas_call(
        body,
        grid=(grid,),
        in_specs=[pl.BlockSpec((NC, blk, CP), lambda i: (0, i, 0)),
                  spec, spec, spec],
        out_specs=[spec, spec],
        out_shape=[jax.ShapeDtypeStruct((n_pad, CP), jnp.float32),
                   jax.ShapeDtypeStruct((n_pad, CP), jnp.float32)],
    )(parts, hs, h0, dinv)


def _log_softmax(h, blk):
    n, c_dim = h.shape

    def body(h_ref, o_ref):
        v = h_ref[...]
        m = jnp.max(v, axis=1, keepdims=True)
        e = jnp.exp(v - m)
        s = jnp.sum(e, axis=1, keepdims=True)
        o_ref[...] = v - m - jnp.log(s)

    grid = n // blk
    spec = pl.BlockSpec((blk, c_dim), lambda i: (i, 0))
    return pl.pallas_call(
        body,
        grid=(grid,),
        in_specs=[spec],
        out_specs=spec,
        out_shape=jax.ShapeDtypeStruct((n, c_dim), jnp.float32),
    )(h)


def kernel(x, edge_index, edge_attr, W1, b1, W2, b2):
    n, f_in = x.shape
    e_tot = edge_attr.shape[0]

    n_pad = ((n + 16 * 64 - 1) // (16 * 64)) * (16 * 64)
    epw = e_tot // NW
    epw_pad = ((epw + WIN - 1) // WIN) * WIN
    nwin = epw_pad // WIN
    pad = epw_pad - epw

    src = edge_index[0].reshape(NW, epw)
    dst = edge_index[1].reshape(NW, epw)
    w = edge_attr.reshape(NW, epw)
    pad_ids = (jnp.arange(pad, dtype=jnp.int32) * 89) % n
    pad_blk = jnp.broadcast_to(pad_ids, (NW, pad))
    srcp = jnp.concatenate([src, pad_blk], axis=1).reshape(NW, nwin, WIN)
    dstp = jnp.concatenate([dst, pad_blk], axis=1).reshape(NW, nwin, WIN)
    wp = jnp.concatenate([w, jnp.zeros((NW, pad), jnp.float32)], axis=1)
    wp = wp.reshape(NW, nwin, WIN)

    zeros_pad = jnp.zeros((n_pad, CP), jnp.float32)
    ones_tab = jnp.pad(jnp.ones((n_pad, CV), jnp.float32), ((0, 0), (0, CP - CV)))

    x_pad = jnp.pad(x, ((0, n_pad - n), (0, 0)))
    W2p = jnp.pad(W2, ((0, CP - CV), (0, 0)))
    b2p = jnp.pad(b2, (0, CP - CV))

    blk = n_pad // 10

    deg_parts = _sc_scatter(ones_tab, srcp, dstp, wp, zeros_pad, n_pad, nwin)
    h0 = _mlp(x_pad, W1, b1, W2p, b2p, n_pad, blk)
    dinv, hs = _prep(deg_parts, h0, n_pad, blk)

    h = h0
    for _ in range(K_ITERS):
        parts = _sc_scatter(hs, srcp, dstp, wp, zeros_pad, n_pad, nwin)
        h, hs = _update(parts, hs, h0, dinv, n_pad, blk)

    return _log_softmax(h[:n, :CV], 1000)


# speedup vs baseline: 14.7652x; 1.3451x over previous
import jax
import jax.numpy as jnp
from jax import lax
from jax.experimental import pallas as pl
from jax.experimental.pallas import tpu as pltpu
from jax.experimental.pallas import tpu_sc as plsc

ALPHA = 0.1
K_ITERS = 10

NC = 2
NS = 16
NW = NC * NS
WIN = 128
CP = 128
CV = 64


def _sc_scatter(table, srcp, dstp, wp, zeros_pad, n_pad, nwin):
    spt = n_pad // NS

    nwh = nwin // 2
    ngrp = nwh // 2

    def body(table_ref, src_ref, dst_ref, w_ref, z_ref, out_ref,
             acc, src_v, dst_v, w_v, rows_a, rows_b,
             sga, sgb, ssa, ssb):
        c = lax.axis_index("c")
        s = lax.axis_index("s")
        wid = c * NS + s
        node0 = s * spt
        pltpu.sync_copy(z_ref.at[pl.ds(node0, spt)], acc.at[pl.ds(node0, spt)])
        plsc.subcore_barrier()

        def g_start(j, rows, sem):
            pltpu.async_copy(table_ref.at[src_v.at[j]], rows, sem)

        def g_wait(rows, sem):
            pltpu.make_async_copy(table_ref.at[src_v.at[0]], rows, sem).wait()

        def s_start(j, rows, sem):
            pltpu.async_copy(rows, acc.at[dst_v.at[j]], sem, add=True)

        def s_wait(rows, sem):
            pltpu.make_async_copy(rows, acc.at[dst_v.at[0]], sem).wait()

        def scale_half(rows, j, h):
            def grp(g, carry2):
                wv = w_v[j, pl.ds(g * 16, 16)]
                for i in range(16):
                    e = g * 16 + i
                    sc = wv[i]
                    for q in range(CV // 16):
                        rows[e, pl.ds(q * 16, 16)] = rows[e, pl.ds(q * 16, 16)] * sc
                return carry2

            nh = WIN // 32
            lax.fori_loop(h * nh, (h + 1) * nh, grp, 0)

        for half in range(2):
            pltpu.sync_copy(src_ref.at[wid].at[pl.ds(half * nwh, nwh)], src_v)
            pltpu.sync_copy(dst_ref.at[wid].at[pl.ds(half * nwh, nwh)], dst_v)
            pltpu.sync_copy(w_ref.at[wid].at[pl.ds(half * nwh, nwh)], w_v)

            g_start(0, rows_a, sga)
            g_start(1, rows_b, sgb)

            def group(t, carry):
                j0 = 2 * t
                g_wait(rows_a, sga)
                scale_half(rows_a, j0, 0)
                @pl.when(t > 0)
                def _():
                    s_wait(rows_b, ssb)
                    g_start(j0 + 1, rows_b, sgb)
                scale_half(rows_a, j0, 1)
                s_start(j0, rows_a, ssa)
                g_wait(rows_b, sgb)
                scale_half(rows_b, j0 + 1, 0)
                @pl.when(t < ngrp - 1)
                def _():
                    s_wait(rows_a, ssa)
                    g_start(j0 + 2, rows_a, sga)
                scale_half(rows_b, j0 + 1, 1)
                s_start(j0 + 1, rows_b, ssb)
                return carry

            lax.fori_loop(0, ngrp, group, 0)
            s_wait(rows_a, ssa)
            s_wait(rows_b, ssb)
        plsc.subcore_barrier()
        pltpu.sync_copy(acc.at[pl.ds(node0, spt)], out_ref.at[c].at[pl.ds(node0, spt)])

    mesh = plsc.VectorSubcoreMesh(core_axis_name="c", subcore_axis_name="s")
    f = pl.kernel(
        body,
        out_type=jax.ShapeDtypeStruct((NC, n_pad, CP), jnp.float32),
        mesh=mesh,
        scratch_types=[
            pltpu.VMEM_SHARED((n_pad, CP), jnp.float32),
            pltpu.VMEM((nwin // 2, WIN), jnp.int32),
            pltpu.VMEM((nwin // 2, WIN), jnp.int32),
            pltpu.VMEM((nwin // 2, WIN), jnp.float32),
            pltpu.VMEM((WIN, CP), jnp.float32),
            pltpu.VMEM((WIN, CP), jnp.float32),
            pltpu.SemaphoreType.DMA,
            pltpu.SemaphoreType.DMA,
            pltpu.SemaphoreType.DMA,
            pltpu.SemaphoreType.DMA,
        ],
    )
    return f(table, srcp, dstp, wp, zeros_pad)


def _mlp(x, W1, b1, W2p, b2p, n_pad, blk):
    f_in = x.shape[1]

    def body(x_ref, w1_ref, b1_ref, w2_ref, b2_ref, o_ref):
        h = jnp.maximum(
            jnp.dot(x_ref[...], w1_ref[...].T, preferred_element_type=jnp.float32)
            + b1_ref[...], 0.0)
        o_ref[...] = (jnp.dot(h, w2_ref[...].T, preferred_element_type=jnp.float32)
                      + b2_ref[...])

    grid = n_pad // blk
    return pl.pallas_call(
        body,
        grid=(grid,),
        in_specs=[
            pl.BlockSpec((blk, f_in), lambda i: (i, 0)),
            pl.BlockSpec(W1.shape, lambda i: (0, 0)),
            pl.BlockSpec((1, W1.shape[0]), lambda i: (0, 0)),
            pl.BlockSpec(W2p.shape, lambda i: (0, 0)),
            pl.BlockSpec((1, CP), lambda i: (0, 0)),
        ],
        out_specs=pl.BlockSpec((blk, CP), lambda i: (i, 0)),
        out_shape=jax.ShapeDtypeStruct((n_pad, CP), jnp.float32),
    )(x, W1, b1.reshape(1, -1), W2p, b2p.reshape(1, -1))


def _prep(parts, h0, n_pad, blk):
    def body(p_ref, h0_ref, dinv_ref, hs_ref):
        deg = p_ref[0, :, 0:1] + p_ref[1, :, 0:1] + 1.0
        dinv = lax.rsqrt(deg)
        dinv_ref[...] = jnp.broadcast_to(dinv, (blk, CP))
        hs_ref[...] = dinv * h0_ref[...]

    grid = n_pad // blk
    spec = pl.BlockSpec((blk, CP), lambda i: (i, 0))
    return pl.pallas_call(
        body,
        grid=(grid,),
        in_specs=[pl.BlockSpec((NC, blk, CP), lambda i: (0, i, 0)), spec],
        out_specs=[spec, spec],
        out_shape=[jax.ShapeDtypeStruct((n_pad, CP), jnp.float32),
                   jax.ShapeDtypeStruct((n_pad, CP), jnp.float32)],
    )(parts, h0)


def _update(parts, hs, h0, dinv, n_pad, blk):
    def body(p_ref, hs_ref, h0_ref, dinv_ref, h_ref, hsn_ref):
        t = p_ref[0] + p_ref[1] + hs_ref[...]
        h_new = (1.0 - ALPHA) * dinv_ref[...] * t + ALPHA * h0_ref[...]
        h_ref[...] = h_new
        hsn_ref[...] = dinv_ref[...] * h_new

    grid = n_pad // blk
    spec = pl.BlockSpec((blk, CP), lambda i: (i, 0))
    return pl.pallas_call(
        body,
        grid=(grid,),
        in_specs=[pl.BlockSpec((NC, blk, CP), lambda i: (0, i, 0)),
                  spec, spec, spec],
        out_specs=[spec, spec],
        out_shape=[jax.ShapeDtypeStruct((n_pad, CP), jnp.float32),
                   jax.ShapeDtypeStruct((n_pad, CP), jnp.float32)],
    )(parts, hs, h0, dinv)


def _log_softmax(h, blk):
    n, c_dim = h.shape

    def body(h_ref, o_ref):
        v = h_ref[...]
        m = jnp.max(v, axis=1, keepdims=True)
        e = jnp.exp(v - m)
        s = jnp.sum(e, axis=1, keepdims=True)
        o_ref[...] = v - m - jnp.log(s)

    grid = n // blk
    spec = pl.BlockSpec((blk, c_dim), lambda i: (i, 0))
    return pl.pallas_call(
        body,
        grid=(grid,),
        in_specs=[spec],
        out_specs=spec,
        out_shape=jax.ShapeDtypeStruct((n, c_dim), jnp.float32),
    )(h)


def kernel(x, edge_index, edge_attr, W1, b1, W2, b2):
    n, f_in = x.shape
    e_tot = edge_attr.shape[0]

    n_pad = ((n + 127) // 128) * 128
    epw = e_tot // NW
    epw_pad = ((epw + 2 * WIN - 1) // (2 * WIN)) * (2 * WIN)
    nwin = epw_pad // WIN
    pad = epw_pad - epw

    src = edge_index[0].reshape(NW, epw)
    dst = edge_index[1].reshape(NW, epw)
    w = edge_attr.reshape(NW, epw)
    pad_ids = (jnp.arange(pad, dtype=jnp.int32) * 89) % n
    pad_blk = jnp.broadcast_to(pad_ids, (NW, pad))
    srcp = jnp.concatenate([src, pad_blk], axis=1).reshape(NW, nwin, WIN)
    dstp = jnp.concatenate([dst, pad_blk], axis=1).reshape(NW, nwin, WIN)
    wp = jnp.concatenate([w, jnp.zeros((NW, pad), jnp.float32)], axis=1)
    wp = wp.reshape(NW, nwin, WIN)

    zeros_pad = jnp.zeros((n_pad, CP), jnp.float32)
    ones_tab = jnp.pad(jnp.ones((n_pad, CV), jnp.float32), ((0, 0), (0, CP - CV)))

    x_pad = jnp.pad(x, ((0, n_pad - n), (0, 0)))
    W2p = jnp.pad(W2, ((0, CP - CV), (0, 0)))
    b2p = jnp.pad(b2, (0, CP - CV))

    blk = n_pad // 8

    deg_parts = _sc_scatter(ones_tab, srcp, dstp, wp, zeros_pad, n_pad, nwin)
    h0 = _mlp(x_pad, W1, b1, W2p, b2p, n_pad, blk)
    dinv, hs = _prep(deg_parts, h0, n_pad, blk)

    h = h0
    for _ in range(K_ITERS):
        parts = _sc_scatter(hs, srcp, dstp, wp, zeros_pad, n_pad, nwin)
        h, hs = _update(parts, hs, h0, dinv, n_pad, blk)

    return _log_softmax(h[:n, :CV], 1000)


# speedup vs baseline: 17.0223x; 1.1529x over previous
import jax
import jax.numpy as jnp
from jax import lax
from jax.experimental import pallas as pl
from jax.experimental.pallas import tpu as pltpu
from jax.experimental.pallas import tpu_sc as plsc

ALPHA = 0.1
K_ITERS = 10

NC = 2
NS = 16
NW = NC * NS
WIN = 128
CP = 128
CV = 64
SRC_BITS = 14


def _sc_scatter(table, packp, wlop, whip, zeros2, n2, nwin):
    spt2 = n2 // NS
    ngrp = nwin // 3

    def body(table_ref, pk_ref, wlo_ref, whi_ref, z_ref, out_ref,
             acc, pk_v, wlo_v, whi_v, rows_a, rows_b, rows_c,
             swa, dwa, swb, dwb, swc, dwc, sga, sgb, sgc, ssa, ssb, ssc):
        c = lax.axis_index("c")
        s = lax.axis_index("s")
        wid = c * NS + s
        pltpu.sync_copy(pk_ref.at[wid], pk_v)
        pltpu.sync_copy(wlo_ref.at[wid], wlo_v)
        pltpu.sync_copy(whi_ref.at[wid], whi_v)
        node0 = s * spt2
        pltpu.sync_copy(z_ref.at[pl.ds(node0, spt2)], acc.at[pl.ds(node0, spt2)])
        plsc.subcore_barrier()

        def unpack(j, sw, dw):
            for g in range(WIN // 16):
                p = pk_v[j, pl.ds(g * 16, 16)]
                sw[pl.ds(g * 16, 16)] = p & ((1 << SRC_BITS) - 1)
                dw[pl.ds(g * 16, 16)] = lax.shift_right_logical(p, SRC_BITS)

        def g_start(rows, sw, sem):
            pltpu.async_copy(table_ref.at[sw], rows, sem)

        def g_wait(rows, sw, sem):
            pltpu.make_async_copy(table_ref.at[sw], rows, sem).wait()

        def s_start(rows, dw, sem):
            pltpu.async_copy(rows, acc.at[dw], sem, add=True)

        def s_wait(rows, dw, sem):
            pltpu.make_async_copy(rows, acc.at[dw], sem).wait()

        def scale(rows, j):
            def grp(g, carry2):
                lo = wlo_v[j, pl.ds(g * 16, 16)]
                hi = whi_v[j, pl.ds(g * 16, 16)]
                for i in range(16):
                    e = g * 16 + i
                    slo = lo[i]
                    shi = hi[i]
                    for q in range(CV // 16):
                        t = rows[e, pl.ds(q * 16, 16)]
                        rows[e, pl.ds(CV + q * 16, 16)] = t * shi
                        rows[e, pl.ds(q * 16, 16)] = t * slo
                return carry2

            lax.fori_loop(0, WIN // 16, grp, 0)

        unpack(0, swa, dwa)
        g_start(rows_a, swa, sga)
        unpack(1, swb, dwb)
        g_start(rows_b, swb, sgb)

        def group(t, carry):
            j0 = 3 * t
            g_wait(rows_a, swa, sga)
            scale(rows_a, j0)
            s_start(rows_a, dwa, ssa)
            @pl.when(t > 0)
            def _():
                s_wait(rows_c, dwc, ssc)
            unpack(j0 + 2, swc, dwc)
            g_start(rows_c, swc, sgc)
            g_wait(rows_b, swb, sgb)
            scale(rows_b, j0 + 1)
            s_start(rows_b, dwb, ssb)
            @pl.when(t < ngrp - 1)
            def _():
                s_wait(rows_a, dwa, ssa)
                unpack(j0 + 3, swa, dwa)
                g_start(rows_a, swa, sga)
            g_wait(rows_c, swc, sgc)
            scale(rows_c, j0 + 2)
            s_start(rows_c, dwc, ssc)
            @pl.when(t < ngrp - 1)
            def _():
                s_wait(rows_b, dwb, ssb)
                unpack(j0 + 4, swb, dwb)
                g_start(rows_b, swb, sgb)
            return carry

        lax.fori_loop(0, ngrp, group, 0)
        s_wait(rows_a, dwa, ssa)
        s_wait(rows_b, dwb, ssb)
        s_wait(rows_c, dwc, ssc)
        plsc.subcore_barrier()
        pltpu.sync_copy(acc.at[pl.ds(node0, spt2)], out_ref.at[c].at[pl.ds(node0, spt2)])

    mesh = plsc.VectorSubcoreMesh(core_axis_name="c", subcore_axis_name="s")
    f = pl.kernel(
        body,
        out_type=jax.ShapeDtypeStruct((NC, n2, CP), jnp.float32),
        mesh=mesh,
        scratch_types=[
            pltpu.VMEM_SHARED((n2, CP), jnp.float32),
            pltpu.VMEM((nwin, WIN), jnp.int32),
            pltpu.VMEM((nwin, WIN), jnp.float32),
            pltpu.VMEM((nwin, WIN), jnp.float32),
            pltpu.VMEM((WIN, CP), jnp.float32),
            pltpu.VMEM((WIN, CP), jnp.float32),
            pltpu.VMEM((WIN, CP), jnp.float32),
            pltpu.VMEM((WIN,), jnp.int32),
            pltpu.VMEM((WIN,), jnp.int32),
            pltpu.VMEM((WIN,), jnp.int32),
            pltpu.VMEM((WIN,), jnp.int32),
            pltpu.VMEM((WIN,), jnp.int32),
            pltpu.VMEM((WIN,), jnp.int32),
            pltpu.SemaphoreType.DMA,
            pltpu.SemaphoreType.DMA,
            pltpu.SemaphoreType.DMA,
            pltpu.SemaphoreType.DMA,
            pltpu.SemaphoreType.DMA,
            pltpu.SemaphoreType.DMA,
        ],
    )
    return f(table, packp, wlop, whip, zeros2)


def _mlp(x, W1, b1, W2p, b2p, n_pad, blk):
    f_in = x.shape[1]

    def body(x_ref, w1_ref, b1_ref, w2_ref, b2_ref, o_ref):
        h = jnp.maximum(
            jnp.dot(x_ref[...], w1_ref[...].T, preferred_element_type=jnp.float32)
            + b1_ref[...], 0.0)
        o_ref[...] = (jnp.dot(h, w2_ref[...].T, preferred_element_type=jnp.float32)
                      + b2_ref[...])

    grid = n_pad // blk
    return pl.pallas_call(
        body,
        grid=(grid,),
        in_specs=[
            pl.BlockSpec((blk, f_in), lambda i: (i, 0)),
            pl.BlockSpec(W1.shape, lambda i: (0, 0)),
            pl.BlockSpec((1, W1.shape[0]), lambda i: (0, 0)),
            pl.BlockSpec(W2p.shape, lambda i: (0, 0)),
            pl.BlockSpec((1, CP), lambda i: (0, 0)),
        ],
        out_specs=pl.BlockSpec((blk, CP), lambda i: (i, 0)),
        out_shape=jax.ShapeDtypeStruct((n_pad, CP), jnp.float32),
    )(x, W1, b1.reshape(1, -1), W2p, b2p.reshape(1, -1))


def _prep(p0, p1, h0, n_pad, blk):
    def body(p0_ref, p1_ref, h0_ref, dinv_ref, hs_ref):
        deg = p0_ref[:, 0:1] + p1_ref[:, 0:1] + 1.0
        dinv = lax.rsqrt(deg)
        dinv_ref[...] = jnp.broadcast_to(dinv, (blk, CV))
        hs_ref[...] = dinv * h0_ref[...]

    grid = n_pad // blk
    specv = pl.BlockSpec((blk, CV), lambda i: (i, 0))
    specp = pl.BlockSpec((blk, CP), lambda i: (i, 0))
    return pl.pallas_call(
        body,
        grid=(grid,),
        in_specs=[specv, specv, specp],
        out_specs=[specv, specp],
        out_shape=[jax.ShapeDtypeStruct((n_pad, CV), jnp.float32),
                   jax.ShapeDtypeStruct((n_pad, CP), jnp.float32)],
    )(p0, p1, h0)


def _update(p0, p1, hs, h0, dinv, n_pad, blk):
    def body(p0_ref, p1_ref, hs_ref, h0_ref, dinv_ref, h_ref, hsn_ref):
        t = p0_ref[...] + p1_ref[...] + hs_ref[:, :CV]
        h_new = (1.0 - ALPHA) * dinv_ref[...] * t + ALPHA * h0_ref[:, :CV]
        h_ref[...] = h_new
        hsn_ref[:, :CV] = dinv_ref[...] * h_new
        hsn_ref[:, CV:] = jnp.zeros((blk, CP - CV), jnp.float32)

    grid = n_pad // blk
    specv = pl.BlockSpec((blk, CV), lambda i: (i, 0))
    specp = pl.BlockSpec((blk, CP), lambda i: (i, 0))
    return pl.pallas_call(
        body,
        grid=(grid,),
        in_specs=[specv, specv, specp, specp, specv],
        out_specs=[specv, specp],
        out_shape=[jax.ShapeDtypeStruct((n_pad, CV), jnp.float32),
                   jax.ShapeDtypeStruct((n_pad, CP), jnp.float32)],
    )(p0, p1, hs, h0, dinv)


def _log_softmax(h, blk):
    n, c_dim = h.shape

    def body(h_ref, o_ref):
        v = h_ref[...]
        m = jnp.max(v, axis=1, keepdims=True)
        e = jnp.exp(v - m)
        s = jnp.sum(e, axis=1, keepdims=True)
        o_ref[...] = v - m - jnp.log(s)

    grid = n // blk
    spec = pl.BlockSpec((blk, c_dim), lambda i: (i, 0))
    return pl.pallas_call(
        body,
        grid=(grid,),
        in_specs=[spec],
        out_specs=spec,
        out_shape=jax.ShapeDtypeStruct((n, c_dim), jnp.float32),
    )(h)


def kernel(x, edge_index, edge_attr, W1, b1, W2, b2):
    n, f_in = x.shape
    e_tot = edge_attr.shape[0]

    n_pad = ((n + 255) // 256) * 256
    n2 = n_pad // 2
    epw = e_tot // NW
    epw_pad = ((epw + 3 * WIN - 1) // (3 * WIN)) * (3 * WIN)
    nwin = epw_pad // WIN
    pad = epw_pad - epw

    src = edge_index[0].reshape(NW, epw)
    dst = edge_index[1].reshape(NW, epw)
    w = edge_attr.reshape(NW, epw)
    pad_ids = (jnp.arange(pad, dtype=jnp.int32) * 89) % n
    pad_blk = jnp.broadcast_to(pad_ids, (NW, pad))
    srcp = jnp.concatenate([src, pad_blk], axis=1)
    dstp = jnp.concatenate([dst, pad_blk], axis=1)
    wp = jnp.concatenate([w, jnp.zeros((NW, pad), jnp.float32)], axis=1)
    par = (dstp & 1).astype(jnp.float32)
    packp = (srcp | ((dstp >> 1) << SRC_BITS)).reshape(NW, nwin, WIN)
    wlop = (wp * (1.0 - par)).reshape(NW, nwin, WIN)
    whip = (wp * par).reshape(NW, nwin, WIN)

    zeros2 = jnp.zeros((n2, CP), jnp.float32)
    ones_tab = jnp.pad(jnp.ones((n_pad, CV), jnp.float32), ((0, 0), (0, CP - CV)))

    x_pad = jnp.pad(x, ((0, n_pad - n), (0, 0)))
    W2p = jnp.pad(W2, ((0, CP - CV), (0, 0)))
    b2p = jnp.pad(b2, (0, CP - CV))

    blk = n_pad // 10

    deg_parts = _sc_scatter(ones_tab, packp, wlop, whip, zeros2, n2, nwin)
    h0 = _mlp(x_pad, W1, b1, W2p, b2p, n_pad, blk)
    dp0 = deg_parts[0].reshape(n_pad, CV)
    dp1 = deg_parts[1].reshape(n_pad, CV)
    dinv, hs = _prep(dp0, dp1, h0, n_pad, blk)

    h = None
    for _ in range(K_ITERS):
        parts = _sc_scatter(hs, packp, wlop, whip, zeros2, n2, nwin)
        p0 = parts[0].reshape(n_pad, CV)
        p1 = parts[1].reshape(n_pad, CV)
        h, hs = _update(p0, p1, hs, h0, dinv, n_pad, blk)

    return _log_softmax(h[:n], 1000)


# speedup vs baseline: 17.3668x; 1.0202x over previous
import jax
import jax.numpy as jnp
from jax import lax
from jax.experimental import pallas as pl
from jax.experimental.pallas import tpu as pltpu
from jax.experimental.pallas import tpu_sc as plsc

ALPHA = 0.1
K_ITERS = 10

NC = 2
NS = 16
NW = NC * NS
WIN = 128
CP = 128
CV = 64
SRC_BITS = 14


def _sc_scatter(table, packp, wlop, whip, zeros2, n2, nwin):
    spt2 = n2 // NS
    ngrp = nwin // 3

    def body(table_ref, pk_ref, wlo_ref, whi_ref, z_ref, out_ref,
             acc, pk_v, wlo_v, whi_v, rows_a, rows_b, rows_c,
             swa, dwa, swb, dwb, swc, dwc, sga, sgb, sgc, ssa, ssb, ssc):
        c = lax.axis_index("c")
        s = lax.axis_index("s")
        wid = c * NS + s
        pltpu.sync_copy(pk_ref.at[wid], pk_v)
        pltpu.sync_copy(wlo_ref.at[wid], wlo_v)
        pltpu.sync_copy(whi_ref.at[wid], whi_v)
        node0 = s * spt2
        pltpu.sync_copy(z_ref.at[pl.ds(node0, spt2)], acc.at[pl.ds(node0, spt2)])
        plsc.subcore_barrier()

        def unpack(j, sw, dw):
            for g in range(WIN // 16):
                p = pk_v[j, pl.ds(g * 16, 16)]
                sw[pl.ds(g * 16, 16)] = p & ((1 << SRC_BITS) - 1)
                dw[pl.ds(g * 16, 16)] = lax.shift_right_logical(p, SRC_BITS)

        def g_start(rows, sw, sem):
            pltpu.async_copy(table_ref.at[sw], rows, sem)

        def g_wait(rows, sw, sem):
            pltpu.make_async_copy(table_ref.at[sw], rows, sem).wait()

        def s_start(rows, dw, sem):
            pltpu.async_copy(rows, acc.at[dw], sem, add=True)

        def s_wait(rows, dw, sem):
            pltpu.make_async_copy(rows, acc.at[dw], sem).wait()

        def scale(rows, j):
            def grp(g, carry2):
                lo = wlo_v[j, pl.ds(g * 16, 16)]
                hi = whi_v[j, pl.ds(g * 16, 16)]
                for i in range(16):
                    e = g * 16 + i
                    slo = lo[i]
                    shi = hi[i]
                    for q in range(CV // 16):
                        t = rows[e, pl.ds(q * 16, 16)]
                        rows[e, pl.ds(CV + q * 16, 16)] = t * shi
                        rows[e, pl.ds(q * 16, 16)] = t * slo
                return carry2

            lax.fori_loop(0, WIN // 16, grp, 0)

        unpack(0, swa, dwa)
        g_start(rows_a, swa, sga)
        unpack(1, swb, dwb)
        g_start(rows_b, swb, sgb)

        def group(t, carry):
            j0 = 3 * t
            g_wait(rows_a, swa, sga)
            scale(rows_a, j0)
            s_start(rows_a, dwa, ssa)
            @pl.when(t > 0)
            def _():
                s_wait(rows_c, dwc, ssc)
            unpack(j0 + 2, swc, dwc)
            g_start(rows_c, swc, sgc)
            g_wait(rows_b, swb, sgb)
            scale(rows_b, j0 + 1)
            s_start(rows_b, dwb, ssb)
            @pl.when(t < ngrp - 1)
            def _():
                s_wait(rows_a, dwa, ssa)
                unpack(j0 + 3, swa, dwa)
                g_start(rows_a, swa, sga)
            g_wait(rows_c, swc, sgc)
            scale(rows_c, j0 + 2)
            s_start(rows_c, dwc, ssc)
            @pl.when(t < ngrp - 1)
            def _():
                s_wait(rows_b, dwb, ssb)
                unpack(j0 + 4, swb, dwb)
                g_start(rows_b, swb, sgb)
            return carry

        lax.fori_loop(0, ngrp, group, 0)
        s_wait(rows_a, dwa, ssa)
        s_wait(rows_b, dwb, ssb)
        s_wait(rows_c, dwc, ssc)
        plsc.subcore_barrier()
        pltpu.sync_copy(acc.at[pl.ds(node0, spt2)], out_ref.at[c].at[pl.ds(node0, spt2)])

    mesh = plsc.VectorSubcoreMesh(core_axis_name="c", subcore_axis_name="s")
    f = pl.kernel(
        body,
        out_type=jax.ShapeDtypeStruct((NC, n2, CP), jnp.float32),
        mesh=mesh,
        scratch_types=[
            pltpu.VMEM_SHARED((n2, CP), jnp.float32),
            pltpu.VMEM((nwin, WIN), jnp.int32),
            pltpu.VMEM((nwin, WIN), jnp.float32),
            pltpu.VMEM((nwin, WIN), jnp.float32),
            pltpu.VMEM((WIN, CP), jnp.float32),
            pltpu.VMEM((WIN, CP), jnp.float32),
            pltpu.VMEM((WIN, CP), jnp.float32),
            pltpu.VMEM((WIN,), jnp.int32),
            pltpu.VMEM((WIN,), jnp.int32),
            pltpu.VMEM((WIN,), jnp.int32),
            pltpu.VMEM((WIN,), jnp.int32),
            pltpu.VMEM((WIN,), jnp.int32),
            pltpu.VMEM((WIN,), jnp.int32),
            pltpu.SemaphoreType.DMA,
            pltpu.SemaphoreType.DMA,
            pltpu.SemaphoreType.DMA,
            pltpu.SemaphoreType.DMA,
            pltpu.SemaphoreType.DMA,
            pltpu.SemaphoreType.DMA,
        ],
    )
    return f(table, packp, wlop, whip, zeros2)


def _mlp(x, W1, b1, W2p, b2p, n_pad, blk):
    f_in = x.shape[1]

    def body(x_ref, w1_ref, b1_ref, w2_ref, b2_ref, o_ref):
        h = jnp.maximum(
            jnp.dot(x_ref[...], w1_ref[...].T, preferred_element_type=jnp.float32)
            + b1_ref[...], 0.0)
        o_ref[...] = (jnp.dot(h, w2_ref[...].T, preferred_element_type=jnp.float32)
                      + b2_ref[...])

    grid = n_pad // blk
    return pl.pallas_call(
        body,
        grid=(grid,),
        in_specs=[
            pl.BlockSpec((blk, f_in), lambda i: (i, 0)),
            pl.BlockSpec(W1.shape, lambda i: (0, 0)),
            pl.BlockSpec((1, W1.shape[0]), lambda i: (0, 0)),
            pl.BlockSpec(W2p.shape, lambda i: (0, 0)),
            pl.BlockSpec((1, CP), lambda i: (0, 0)),
        ],
        out_specs=pl.BlockSpec((blk, CP), lambda i: (i, 0)),
        out_shape=jax.ShapeDtypeStruct((n_pad, CP), jnp.float32),
    )(x, W1, b1.reshape(1, -1), W2p, b2p.reshape(1, -1))


def _prep(p0, p1, h0, n_pad, blk):
    def body(p0_ref, p1_ref, h0_ref, dinv_ref, hs_ref):
        deg = p0_ref[:, 0:1] + p1_ref[:, 0:1] + 1.0
        dinv = lax.rsqrt(deg)
        dinv_ref[...] = jnp.broadcast_to(dinv, (blk, CV))
        hs_ref[...] = dinv * h0_ref[...]

    grid = n_pad // blk
    specv = pl.BlockSpec((blk, CV), lambda i: (i, 0))
    specp = pl.BlockSpec((blk, CP), lambda i: (i, 0))
    return pl.pallas_call(
        body,
        grid=(grid,),
        in_specs=[specv, specv, specp],
        out_specs=[specv, specp],
        out_shape=[jax.ShapeDtypeStruct((n_pad, CV), jnp.float32),
                   jax.ShapeDtypeStruct((n_pad, CP), jnp.float32)],
    )(p0, p1, h0)


def _update(p0, p1, hs, h0, dinv, n_pad, blk):
    def body(p0_ref, p1_ref, hs_ref, h0_ref, dinv_ref, hsn_ref):
        t = p0_ref[...] + p1_ref[...] + hs_ref[:, :CV]
        h_new = (1.0 - ALPHA) * dinv_ref[...] * t + ALPHA * h0_ref[:, :CV]
        hsn_ref[:, :CV] = dinv_ref[...] * h_new
        hsn_ref[:, CV:] = jnp.zeros((blk, CP - CV), jnp.float32)

    grid = n_pad // blk
    specv = pl.BlockSpec((blk, CV), lambda i: (i, 0))
    specp = pl.BlockSpec((blk, CP), lambda i: (i, 0))
    return pl.pallas_call(
        body,
        grid=(grid,),
        in_specs=[specv, specv, specp, specp, specv],
        out_specs=specp,
        out_shape=jax.ShapeDtypeStruct((n_pad, CP), jnp.float32),
    )(p0, p1, hs, h0, dinv)


def _update_last(p0, p1, hs, h0, dinv, n_pad, blk):
    def body(p0_ref, p1_ref, hs_ref, h0_ref, dinv_ref, o_ref):
        t = p0_ref[...] + p1_ref[...] + hs_ref[:, :CV]
        v = (1.0 - ALPHA) * dinv_ref[...] * t + ALPHA * h0_ref[:, :CV]
        m = jnp.max(v, axis=1, keepdims=True)
        e = jnp.exp(v - m)
        s = jnp.sum(e, axis=1, keepdims=True)
        o_ref[...] = v - m - jnp.log(s)

    grid = n_pad // blk
    specv = pl.BlockSpec((blk, CV), lambda i: (i, 0))
    specp = pl.BlockSpec((blk, CP), lambda i: (i, 0))
    return pl.pallas_call(
        body,
        grid=(grid,),
        in_specs=[specv, specv, specp, specp, specv],
        out_specs=specv,
        out_shape=jax.ShapeDtypeStruct((n_pad, CV), jnp.float32),
    )(p0, p1, hs, h0, dinv)


def kernel(x, edge_index, edge_attr, W1, b1, W2, b2):
    n, f_in = x.shape
    e_tot = edge_attr.shape[0]

    n_pad = ((n + 255) // 256) * 256
    n2 = n_pad // 2
    epw = e_tot // NW
    epw_pad = ((epw + 3 * WIN - 1) // (3 * WIN)) * (3 * WIN)
    nwin = epw_pad // WIN
    pad = epw_pad - epw

    src = edge_index[0].reshape(NW, epw)
    dst = edge_index[1].reshape(NW, epw)
    w = edge_attr.reshape(NW, epw)
    pad_ids = (jnp.arange(pad, dtype=jnp.int32) * 89) % n
    pad_blk = jnp.broadcast_to(pad_ids, (NW, pad))
    srcp = jnp.concatenate([src, pad_blk], axis=1)
    dstp = jnp.concatenate([dst, pad_blk], axis=1)
    wp = jnp.concatenate([w, jnp.zeros((NW, pad), jnp.float32)], axis=1)
    par = (dstp & 1).astype(jnp.float32)
    packp = (srcp | ((dstp >> 1) << SRC_BITS)).reshape(NW, nwin, WIN)
    wlop = (wp * (1.0 - par)).reshape(NW, nwin, WIN)
    whip = (wp * par).reshape(NW, nwin, WIN)

    zeros2 = jnp.zeros((n2, CP), jnp.float32)
    ones_tab = jnp.pad(jnp.ones((n_pad, CV), jnp.float32), ((0, 0), (0, CP - CV)))

    x_pad = jnp.pad(x, ((0, n_pad - n), (0, 0)))
    W2p = jnp.pad(W2, ((0, CP - CV), (0, 0)))
    b2p = jnp.pad(b2, (0, CP - CV))

    blk = n_pad // 10

    deg_parts = _sc_scatter(ones_tab, packp, wlop, whip, zeros2, n2, nwin)
    h0 = _mlp(x_pad, W1, b1, W2p, b2p, n_pad, blk)
    dp0 = deg_parts[0].reshape(n_pad, CV)
    dp1 = deg_parts[1].reshape(n_pad, CV)
    dinv, hs = _prep(dp0, dp1, h0, n_pad, blk)

    for _ in range(K_ITERS - 1):
        parts = _sc_scatter(hs, packp, wlop, whip, zeros2, n2, nwin)
        p0 = parts[0].reshape(n_pad, CV)
        p1 = parts[1].reshape(n_pad, CV)
        hs = _update(p0, p1, hs, h0, dinv, n_pad, blk)

    parts = _sc_scatter(hs, packp, wlop, whip, zeros2, n2, nwin)
    p0 = parts[0].reshape(n_pad, CV)
    p1 = parts[1].reshape(n_pad, CV)
    out = _update_last(p0, p1, hs, h0, dinv, n_pad, blk)
    return out[:n]
